# Initial kernel scaffold; baseline (speedup 1.0000x reference)
import jax
import jax.numpy as jnp
from jax.experimental import pallas as pl


def kernel(x, senders, receivers, n_node, weight, bias):
    raise NotImplementedError("write your pallas kernel here")


# speedup vs baseline: 3.0528x; 3.0528x over previous
import functools

import jax
import jax.numpy as jnp
from jax import lax
from jax.experimental import pallas as pl
from jax.experimental.pallas import tpu as pltpu
from jax.experimental.pallas import tpu_sc as plsc

NC = 2
NS = 16
NW = NC * NS
L = 16


@functools.lru_cache(maxsize=None)
def _make_degree_kernel(E_W: int, N_R: int):
    chunk = N_R // NS
    mesh = plsc.VectorSubcoreMesh(core_axis_name="c", subcore_axis_name="s")

    @functools.partial(
        pl.kernel,
        mesh=mesh,
        out_type=jax.ShapeDtypeStruct((NC, 2, N_R), jnp.float32),
        scratch_types=[
            pltpu.VMEM((E_W,), jnp.int32),
            pltpu.VMEM((E_W,), jnp.int32),
            pltpu.VMEM((N_R,), jnp.float32),
            pltpu.VMEM((N_R,), jnp.float32),
            pltpu.VMEM((chunk,), jnp.float32),
            pltpu.VMEM((chunk,), jnp.float32),
            pltpu.VMEM_SHARED((NS, 2, N_R), jnp.float32),
        ],
        compiler_params=pltpu.CompilerParams(needs_layout_passes=False),
    )
    def deg_k(s_hbm, r_hbm, out_hbm, sidx_v, ridx_v, acc_s, acc_r, tmp_v,
              acc2_v, shared):
        c = lax.axis_index("c")
        s = lax.axis_index("s")
        wid = c * NS + s
        pltpu.sync_copy(s_hbm.at[pl.ds(wid * E_W, E_W)], sidx_v)
        pltpu.sync_copy(r_hbm.at[pl.ds(wid * E_W, E_W)], ridx_v)

        zeros = jnp.zeros((L,), jnp.float32)
        ones = jnp.ones((L,), jnp.float32)

        def zero_body(i, _):
            acc_s[pl.ds(i * L, L)] = zeros
            acc_r[pl.ds(i * L, L)] = zeros
            return 0
        lax.fori_loop(0, N_R // L, zero_body, 0)

        def hist_body(i, _):
            si = sidx_v[pl.ds(i * L, L)]
            plsc.addupdate_scatter(acc_s, [si], ones)
            ri = ridx_v[pl.ds(i * L, L)]
            plsc.addupdate_scatter(acc_r, [ri], ones)
            return 0
        lax.fori_loop(0, E_W // L, hist_body, 0)

        pltpu.sync_copy(acc_s, shared.at[s, 0])
        pltpu.sync_copy(acc_r, shared.at[s, 1])
        plsc.subcore_barrier()

        col = s * chunk
        for a in range(2):
            pltpu.sync_copy(shared.at[0, a, pl.ds(col, chunk)], acc2_v)
            for j in range(1, NS):
                pltpu.sync_copy(shared.at[j, a, pl.ds(col, chunk)], tmp_v)

                def add_body(i, _):
                    acc2_v[pl.ds(i * L, L)] = (
                        acc2_v[pl.ds(i * L, L)] + tmp_v[pl.ds(i * L, L)])
                    return 0
                lax.fori_loop(0, chunk // L, add_body, 0)
            pltpu.sync_copy(acc2_v, out_hbm.at[c, a, pl.ds(col, chunk)])

    return deg_k


@functools.lru_cache(maxsize=None)
def _make_edge_kernel(CH: int, K: int, N_R: int, D: int):
    chunk = N_R // NS
    NH = 2
    CHH = CH // NH
    mesh = plsc.VectorSubcoreMesh(core_axis_name="c", subcore_axis_name="s")

    @functools.partial(
        pl.kernel,
        mesh=mesh,
        out_type=jax.ShapeDtypeStruct((NC, N_R, D), jnp.float32),
        scratch_types=[
            pltpu.VMEM((CHH, K), jnp.int32),
            pltpu.VMEM((CHH, K), jnp.int32),
            pltpu.VMEM((2, K, D), jnp.float32),
            pltpu.VMEM_SHARED((N_R, D), jnp.float32),
            pltpu.SemaphoreType.DMA,
            pltpu.SemaphoreType.DMA,
        ],
        compiler_params=pltpu.CompilerParams(needs_layout_passes=False),
    )
    def edge_k(h_hbm, sidx_hbm, ridx_hbm, out_hbm, sidx_v, ridx_v, rows_v,
               acc_sh, sem0, sem1):
        c = lax.axis_index("c")
        s = lax.axis_index("s")
        wid = c * NS + s

        zeros = jnp.zeros((L,), jnp.float32)
        dl = D // L

        def zero_body(i, _):
            rows_v[0, i // dl, pl.ds((i % dl) * L, L)] = zeros
            return 0
        lax.fori_loop(0, K * dl, zero_body, 0)
        for b in range(chunk // K):
            pltpu.sync_copy(rows_v.at[0], acc_sh.at[pl.ds(s * chunk + b * K, K)])
        plsc.subcore_barrier()

        sems = [sem0, sem1]

        def start_gather(chu, b):
            pltpu.make_async_copy(
                h_hbm.at[sidx_v.at[chu]], rows_v.at[b], sems[b]).start()

        def wait_gather(b):
            pltpu.make_async_copy(
                h_hbm.at[sidx_v.at[0]], rows_v.at[b], sems[b]).wait()

        for half in range(NH):
            pltpu.sync_copy(sidx_hbm.at[wid, pl.ds(half * CHH, CHH)], sidx_v)
            pltpu.sync_copy(ridx_hbm.at[wid, pl.ds(half * CHH, CHH)], ridx_v)
            start_gather(0, 0)

            def outer(g, _):
                for b in range(2):
                    chu = g * 2 + b
                    wait_gather(b)
                    nxt = chu + 1

                    @pl.when(nxt < CHH)
                    def _():
                        start_gather(nxt, 1 - b)

                    pltpu.sync_copy(rows_v.at[b], acc_sh.at[ridx_v.at[chu]],
                                    add=True)
                return 0
            lax.fori_loop(0, CHH // 2, outer, 0)

        plsc.subcore_barrier()
        pltpu.sync_copy(acc_sh.at[pl.ds(s * chunk, chunk)],
                        out_hbm.at[c, pl.ds(s * chunk, chunk)])

    return edge_k


def _mm_body(x_ref, w_ref, b_ref, d_ref, o_ref):
    h = jnp.dot(x_ref[...], w_ref[...], preferred_element_type=jnp.float32)
    h = h + b_ref[...]
    deg = d_ref[0] + d_ref[1]
    o_ref[...] = h * lax.rsqrt(jnp.maximum(deg, 1.0))


def _comb_body(p_ref, d_ref, o_ref):
    ssum = p_ref[0] + p_ref[1]
    deg = d_ref[0] + d_ref[1]
    o_ref[...] = ssum * lax.rsqrt(jnp.maximum(deg, 1.0))


def kernel(x, senders, receivers, n_node, weight, bias):
    N, D_IN = x.shape
    D_OUT = weight.shape[1]
    E = senders.shape[0]

    N_R = -(-(N + 1) // (NS * 128)) * (NS * 128)
    K = 128
    CH = -(-(-(-E // (NW * K))) // 4) * 4
    EP = NW * CH * K

    pad = EP - E
    sp = jnp.concatenate([senders, jnp.zeros((pad,), jnp.int32)])
    rp = jnp.concatenate([receivers, jnp.full((pad,), N, jnp.int32)])
    sp = sp.reshape(NW, CH, K)
    rp = rp.reshape(NW, CH, K)

    dpart = _make_degree_kernel(E // NW, N_R)(senders, receivers)
    deg_s = dpart[:, 0, :N, None]
    deg_r = dpart[:, 1, :N, None]

    bm = 2000
    grid = (N // bm,)
    h = pl.pallas_call(
        _mm_body,
        grid=grid,
        in_specs=[
            pl.BlockSpec((bm, D_IN), lambda i: (i, 0)),
            pl.BlockSpec((D_IN, D_OUT), lambda i: (0, 0)),
            pl.BlockSpec((1, D_OUT), lambda i: (0, 0)),
            pl.BlockSpec((NC, bm, 1), lambda i: (0, i, 0)),
        ],
        out_specs=pl.BlockSpec((bm, D_OUT), lambda i: (i, 0)),
        out_shape=jax.ShapeDtypeStruct((N, D_OUT), jnp.float32),
    )(x, weight, bias.reshape(1, D_OUT), deg_s)

    parts = _make_edge_kernel(CH, K, N_R, D_OUT)(h, sp, rp)

    out = pl.pallas_call(
        _comb_body,
        grid=grid,
        in_specs=[
            pl.BlockSpec((NC, bm, D_OUT), lambda i: (0, i, 0)),
            pl.BlockSpec((NC, bm, 1), lambda i: (0, i, 0)),
        ],
        out_specs=pl.BlockSpec((bm, D_OUT), lambda i: (i, 0)),
        out_shape=jax.ShapeDtypeStruct((N, D_OUT), jnp.float32),
    )(parts, deg_r)
    return out


# speedup vs baseline: 3.2277x; 1.0573x over previous
import functools

import jax
import jax.numpy as jnp
from jax import lax
from jax.experimental import pallas as pl
from jax.experimental.pallas import tpu as pltpu
from jax.experimental.pallas import tpu_sc as plsc

NC = 2
NS = 16
NW = NC * NS
L = 16


@functools.lru_cache(maxsize=None)
def _make_degree_kernel(E_W: int, N_R: int):
    chunk = N_R // NS
    mesh = plsc.VectorSubcoreMesh(core_axis_name="c", subcore_axis_name="s")

    @functools.partial(
        pl.kernel,
        mesh=mesh,
        out_type=jax.ShapeDtypeStruct((NC, 2, N_R), jnp.float32),
        scratch_types=[
            pltpu.VMEM((E_W,), jnp.int32),
            pltpu.VMEM((E_W,), jnp.int32),
            pltpu.VMEM((N_R,), jnp.float32),
            pltpu.VMEM((N_R,), jnp.float32),
            pltpu.VMEM((chunk,), jnp.float32),
            pltpu.VMEM((chunk,), jnp.float32),
            pltpu.VMEM_SHARED((NS, 2, N_R), jnp.float32),
        ],
        compiler_params=pltpu.CompilerParams(needs_layout_passes=False),
    )
    def deg_k(s_hbm, r_hbm, out_hbm, sidx_v, ridx_v, acc_s, acc_r, tmp_v,
              acc2_v, shared):
        c = lax.axis_index("c")
        s = lax.axis_index("s")
        wid = c * NS + s
        pltpu.sync_copy(s_hbm.at[pl.ds(wid * E_W, E_W)], sidx_v)
        pltpu.sync_copy(r_hbm.at[pl.ds(wid * E_W, E_W)], ridx_v)

        zeros = jnp.zeros((L,), jnp.float32)
        ones = jnp.ones((L,), jnp.float32)

        def zero_body(i, _):
            acc_s[pl.ds(i * L, L)] = zeros
            acc_r[pl.ds(i * L, L)] = zeros
            return 0
        lax.fori_loop(0, N_R // L, zero_body, 0)

        def hist_body(i, _):
            si = sidx_v[pl.ds(i * L, L)]
            plsc.addupdate_scatter(acc_s, [si], ones)
            ri = ridx_v[pl.ds(i * L, L)]
            plsc.addupdate_scatter(acc_r, [ri], ones)
            return 0
        lax.fori_loop(0, E_W // L, hist_body, 0)

        pltpu.sync_copy(acc_s, shared.at[s, 0])
        pltpu.sync_copy(acc_r, shared.at[s, 1])
        plsc.subcore_barrier()

        col = s * chunk
        for a in range(2):
            pltpu.sync_copy(shared.at[0, a, pl.ds(col, chunk)], acc2_v)
            for j in range(1, NS):
                pltpu.sync_copy(shared.at[j, a, pl.ds(col, chunk)], tmp_v)

                def add_body(i, _):
                    acc2_v[pl.ds(i * L, L)] = (
                        acc2_v[pl.ds(i * L, L)] + tmp_v[pl.ds(i * L, L)])
                    return 0
                lax.fori_loop(0, chunk // L, add_body, 0)
            pltpu.sync_copy(acc2_v, out_hbm.at[c, a, pl.ds(col, chunk)])

    return deg_k


@functools.lru_cache(maxsize=None)
def _make_edge_kernel(CH: int, K: int, N_R: int, D: int):
    chunk = N_R // NS
    NH = 2
    CHH = CH // NH
    mesh = plsc.VectorSubcoreMesh(core_axis_name="c", subcore_axis_name="s")

    @functools.partial(
        pl.kernel,
        mesh=mesh,
        out_type=jax.ShapeDtypeStruct((NC, N_R, D), jnp.float32),
        scratch_types=[
            pltpu.VMEM((CHH, K), jnp.int32),
            pltpu.VMEM((CHH, K), jnp.int32),
            pltpu.VMEM((2, K, D), jnp.float32),
            pltpu.VMEM_SHARED((N_R, D), jnp.float32),
            pltpu.SemaphoreType.DMA,
            pltpu.SemaphoreType.DMA,
        ],
        compiler_params=pltpu.CompilerParams(needs_layout_passes=False),
    )
    def edge_k(h_hbm, sidx_hbm, ridx_hbm, out_hbm, sidx_v, ridx_v, rows_v,
               acc_sh, sem0, sem1):
        c = lax.axis_index("c")
        s = lax.axis_index("s")
        wid = c * NS + s
        h_hbm = h_hbm.at[c]

        zeros = jnp.zeros((L,), jnp.float32)
        dl = D // L

        def zero_body(i, _):
            rows_v[0, i // dl, pl.ds((i % dl) * L, L)] = zeros
            return 0
        lax.fori_loop(0, K * dl, zero_body, 0)
        for b in range(chunk // K):
            pltpu.sync_copy(rows_v.at[0], acc_sh.at[pl.ds(s * chunk + b * K, K)])
        plsc.subcore_barrier()

        sems = [sem0, sem1]

        def start_gather(chu, b):
            pltpu.make_async_copy(
                h_hbm.at[sidx_v.at[chu]], rows_v.at[b], sems[b]).start()

        def wait_gather(b):
            pltpu.make_async_copy(
                h_hbm.at[sidx_v.at[0]], rows_v.at[b], sems[b]).wait()

        for half in range(NH):
            pltpu.sync_copy(sidx_hbm.at[wid, pl.ds(half * CHH, CHH)], sidx_v)
            pltpu.sync_copy(ridx_hbm.at[wid, pl.ds(half * CHH, CHH)], ridx_v)
            start_gather(0, 0)

            def outer(g, _):
                for b in range(2):
                    chu = g * 2 + b
                    wait_gather(b)
                    nxt = chu + 1

                    @pl.when(nxt < CHH)
                    def _():
                        start_gather(nxt, 1 - b)

                    pltpu.sync_copy(rows_v.at[b], acc_sh.at[ridx_v.at[chu]],
                                    add=True)
                return 0
            lax.fori_loop(0, CHH // 2, outer, 0)

        plsc.subcore_barrier()
        pltpu.sync_copy(acc_sh.at[pl.ds(s * chunk, chunk)],
                        out_hbm.at[c, pl.ds(s * chunk, chunk)])

    return edge_k


def _mm_body(x_ref, w_ref, b_ref, d_ref, o_ref):
    h = jnp.dot(x_ref[...], w_ref[...], preferred_element_type=jnp.float32)
    h = h + b_ref[...]
    deg = d_ref[0] + d_ref[1]
    hs = h * lax.rsqrt(jnp.maximum(deg, 1.0))
    o_ref[0] = hs
    o_ref[1] = hs


def _comb_body(p_ref, d_ref, o_ref):
    ssum = p_ref[0] + p_ref[1]
    deg = d_ref[0] + d_ref[1]
    o_ref[...] = ssum * lax.rsqrt(jnp.maximum(deg, 1.0))


def kernel(x, senders, receivers, n_node, weight, bias):
    N, D_IN = x.shape
    D_OUT = weight.shape[1]
    E = senders.shape[0]

    N_R = -(-(N + 1) // (NS * 128)) * (NS * 128)
    K = 128
    CH = -(-(-(-E // (NW * K))) // 4) * 4
    EP = NW * CH * K

    pad = EP - E
    sp = jnp.concatenate([senders, jnp.zeros((pad,), jnp.int32)])
    rp = jnp.concatenate([receivers, jnp.full((pad,), N, jnp.int32)])
    sp = sp.reshape(NW, CH, K)
    rp = rp.reshape(NW, CH, K)

    dpart = _make_degree_kernel(E // NW, N_R)(senders, receivers)
    deg_s = dpart[:, 0, :N, None]
    deg_r = dpart[:, 1, :N, None]

    bm = 2000
    grid = (N // bm,)
    h = pl.pallas_call(
        _mm_body,
        grid=grid,
        in_specs=[
            pl.BlockSpec((bm, D_IN), lambda i: (i, 0)),
            pl.BlockSpec((D_IN, D_OUT), lambda i: (0, 0)),
            pl.BlockSpec((1, D_OUT), lambda i: (0, 0)),
            pl.BlockSpec((NC, bm, 1), lambda i: (0, i, 0)),
        ],
        out_specs=pl.BlockSpec((NC, bm, D_OUT), lambda i: (0, i, 0)),
        out_shape=jax.ShapeDtypeStruct((NC, N, D_OUT), jnp.float32),
    )(x, weight, bias.reshape(1, D_OUT), deg_s)

    parts = _make_edge_kernel(CH, K, N_R, D_OUT)(h, sp, rp)

    out = pl.pallas_call(
        _comb_body,
        grid=grid,
        in_specs=[
            pl.BlockSpec((NC, bm, D_OUT), lambda i: (0, i, 0)),
            pl.BlockSpec((NC, bm, 1), lambda i: (0, i, 0)),
        ],
        out_specs=pl.BlockSpec((bm, D_OUT), lambda i: (i, 0)),
        out_shape=jax.ShapeDtypeStruct((N, D_OUT), jnp.float32),
    )(parts, deg_r)
    return out
